# baseline (speedup 1.0000x reference)
import jax
import jax.numpy as jnp
from jax import lax
from jax.experimental import pallas as pl
from jax.experimental.pallas import tpu as pltpu
from jax.experimental.pallas import tpu_sc as plsc

N_NODES = 10000
N_EDGES = 320000
D = 128
NPW = 16
NC = 2
NS = 16
NW = NC * NS
EPT = N_EDGES // NW
CHUNK = 80
NCHUNK = EPT // CHUNK
RPT = N_NODES // NS
ZROWS = 125
DEN_ROWS = N_NODES // 2
DRPT = 312
BM = 1000


def _build_tables(h, hn_ref, np_ref):
    nrm = jnp.sqrt(jnp.sum(h * h, axis=1, keepdims=True)) + 1e-8
    hn_ref[...] = h / nrm
    lanes = lax.broadcasted_iota(jnp.int32, (h.shape[0], NPW), 1)
    np_ref[...] = jnp.where(lanes == 0, nrm, 0.0)


def _proj_table_kernel(x_ref, w_ref, b_ref, hn_ref, np_ref):
    h = jnp.dot(x_ref[...], w_ref[...],
                preferred_element_type=jnp.float32) + b_ref[...]
    _build_tables(h, hn_ref, np_ref)


def _combine(pn0, pn1, pd0, pd1):
    numer = pn0 + pn1
    den = (pd0 + pd1)[:, 0:1]
    return jnp.maximum(numer / (den + 1e-16), 0.0)


def _combine_proj_kernel(pn0_ref, pn1_ref, pd0_ref, pd1_ref, w_ref, b_ref,
                         hn_ref, np_ref):
    h1 = _combine(pn0_ref[...], pn1_ref[...], pd0_ref[...], pd1_ref[...])
    h = jnp.dot(h1, w_ref[...],
                preferred_element_type=jnp.float32) + b_ref[...]
    _build_tables(h, hn_ref, np_ref)


def _combine_out_kernel(pn0_ref, pn1_ref, pd0_ref, pd1_ref, out_ref):
    out_ref[...] = _combine(pn0_ref[...], pn1_ref[...],
                            pd0_ref[...], pd1_ref[...])


_TAB_OUT = (
    jax.ShapeDtypeStruct((N_NODES, D), jnp.float32),
    jax.ShapeDtypeStruct((N_NODES, NPW), jnp.float32),
)
_TAB_OUT_SPECS = (
    pl.BlockSpec((BM, D), lambda i: (i, 0)),
    pl.BlockSpec((BM, NPW), lambda i: (i, 0)),
)


def _proj_call(x, W, b):
    return pl.pallas_call(
        _proj_table_kernel,
        grid=(N_NODES // BM,),
        in_specs=[
            pl.BlockSpec((BM, D), lambda i: (i, 0)),
            pl.BlockSpec((D, D), lambda i: (0, 0)),
            pl.BlockSpec((1, D), lambda i: (0, 0)),
        ],
        out_specs=_TAB_OUT_SPECS,
        out_shape=_TAB_OUT,
    )(x, W, b.reshape(1, D))


def _combine_proj_call(pn0, pn1, pd0, pd1, W, b):
    return pl.pallas_call(
        _combine_proj_kernel,
        grid=(N_NODES // BM,),
        in_specs=[
            pl.BlockSpec((BM, D), lambda i: (i, 0)),
            pl.BlockSpec((BM, D), lambda i: (i, 0)),
            pl.BlockSpec((BM, 8), lambda i: (i, 0)),
            pl.BlockSpec((BM, 8), lambda i: (i, 0)),
            pl.BlockSpec((D, D), lambda i: (0, 0)),
            pl.BlockSpec((1, D), lambda i: (0, 0)),
        ],
        out_specs=_TAB_OUT_SPECS,
        out_shape=_TAB_OUT,
    )(pn0, pn1, pd0, pd1, W, b.reshape(1, D))


def _combine_out_call(pn0, pn1, pd0, pd1):
    return pl.pallas_call(
        _combine_out_kernel,
        grid=(N_NODES // BM,),
        in_specs=[
            pl.BlockSpec((BM, D), lambda i: (i, 0)),
            pl.BlockSpec((BM, D), lambda i: (i, 0)),
            pl.BlockSpec((BM, 8), lambda i: (i, 0)),
            pl.BlockSpec((BM, 8), lambda i: (i, 0)),
        ],
        out_specs=pl.BlockSpec((BM, D), lambda i: (i, 0)),
        out_shape=jax.ShapeDtypeStruct((N_NODES, D), jnp.float32),
    )(pn0, pn1, pd0, pd1)


def _edge_kernel_body(hn_tab, np_tab, srce, dste, betav, out_n, out_d,
                      srcbuf0, dstbuf0, npgbuf0, sidx0, didx0,
                      srcbuf1, dstbuf1, npgbuf1, sidx1, didx1,
                      denbuf, didx2, parbuf, betabuf, sbuf, wbuf,
                      numer, den,
                      gs0, gd0, gn0, gs1, gd1, gn1, is0, id0, is1, id1):
    c = lax.axis_index("c")
    s = lax.axis_index("s")
    tile_base = (c * NS + s) * EPT
    lane = lax.iota(jnp.int32, 16)
    SETS = (
        (srcbuf0, dstbuf0, npgbuf0, sidx0, didx0, gs0, gd0, gn0, is0, id0),
        (srcbuf1, dstbuf1, npgbuf1, sidx1, didx1, gs1, gd1, gn1, is1, id1),
    )

    pltpu.sync_copy(betav, betabuf)
    beta = betabuf[...]

    def _zero_body(i, carry):
        srcbuf0[i // (D // 16), pl.ds((i % (D // 16)) * 16, 16)] = (
            jnp.zeros((16,), jnp.float32))
        return carry
    lax.fori_loop(0, CHUNK * (D // 16), _zero_body, 0)

    def _dzero_body(i, carry):
        denbuf[i, pl.ds(0, 16)] = jnp.zeros((16,), jnp.float32)
        return carry
    lax.fori_loop(0, CHUNK, _dzero_body, 0)

    for k in range(RPT // CHUNK):
        pltpu.sync_copy(srcbuf0,
                        numer.at[pl.ds(s * RPT + k * CHUNK, CHUNK)])
    pltpu.sync_copy(srcbuf0.at[pl.ds(0, RPT % CHUNK)],
                    numer.at[pl.ds(s * RPT + (RPT // CHUNK) * CHUNK,
                                   RPT % CHUNK)])
    for k in range(DRPT // CHUNK):
        pltpu.sync_copy(denbuf, den.at[pl.ds(s * DRPT + k * CHUNK, CHUNK)])
    pltpu.sync_copy(denbuf.at[pl.ds(0, DRPT % CHUNK)],
                    den.at[pl.ds(s * DRPT + (DRPT // CHUNK) * CHUNK,
                                 DRPT % CHUNK)])

    @pl.when(s == 0)
    def _zero_den_tail():
        pltpu.sync_copy(denbuf.at[pl.ds(0, DEN_ROWS - NS * DRPT)],
                        den.at[pl.ds(NS * DRPT, DEN_ROWS - NS * DRPT)])

    plsc.subcore_barrier()


    def ebase(cv):
        return tile_base + cv * CHUNK

    def issue_idx(cv, p):
        _, _, _, si, di, _, _, _, i_s, i_d = SETS[p]
        pltpu.async_copy(srce.at[pl.ds(ebase(cv), CHUNK)], si, i_s)
        pltpu.async_copy(dste.at[pl.ds(ebase(cv), CHUNK)], di, i_d)

    def wait_idx(cv, p):
        _, _, _, si, di, _, _, _, i_s, i_d = SETS[p]
        pltpu.make_async_copy(srce.at[pl.ds(ebase(cv), CHUNK)], si, i_s).wait()
        pltpu.make_async_copy(dste.at[pl.ds(ebase(cv), CHUNK)], di, i_d).wait()

    def issue_gathers(p):
        sb, db, nb, si, di, g_s, g_d, g_n, _, _ = SETS[p]
        pltpu.async_copy(hn_tab.at[si], sb, g_s)
        pltpu.async_copy(hn_tab.at[di], db, g_d)
        pltpu.async_copy(np_tab.at[si], nb, g_n)

    def wait_gathers(p):
        sb, db, nb, si, di, g_s, g_d, g_n, _, _ = SETS[p]
        pltpu.make_async_copy(hn_tab.at[si], sb, g_s).wait()
        pltpu.make_async_copy(hn_tab.at[di], db, g_d).wait()
        pltpu.make_async_copy(np_tab.at[si], nb, g_n).wait()

    def compute_scatter(p):
        sb, db, nb, si, di, _, _, _, _, _ = SETS[p]

        for g in range(CHUNK // 16):
            dvi = di[pl.ds(g * 16, 16)]
            didx2[pl.ds(g * 16, 16)] = lax.shift_right_logical(dvi, 1)
            parbuf[pl.ds(g * 16, 16)] = (dvi & 1) * 8

        for g in range(CHUNK // 16):
            def dot16(i4, dvs, g=g):
                out = []
                for q in range(4):
                    e16 = i4 * 4 + q
                    e = g * 16 + e16
                    acc = sb[e, pl.ds(0, 16)] * db[e, pl.ds(0, 16)]
                    for k in range(1, D // 16):
                        acc = acc + (sb[e, pl.ds(k * 16, 16)] *
                                     db[e, pl.ds(k * 16, 16)])
                    out.append(jnp.where(lane == e16, jnp.sum(acc), dvs[q]))
                return tuple(out)
            z16 = jnp.zeros((16,), jnp.float32)
            dvs = lax.fori_loop(0, 4, dot16, (z16, z16, z16, z16))
            dv = (dvs[0] + dvs[1]) + (dvs[2] + dvs[3])
            rows = lane + g * 16
            npv = plsc.load_gather(
                nb, [rows, jnp.zeros((16,), jnp.int32)])
            wv = jnp.exp(beta * dv)
            sbuf[pl.ds(g * 16, 16)] = wv * npv
            wbuf[pl.ds(g * 16, 16)] = wv

        def scale_body(e, cc):
            se = sbuf[pl.ds(e, 16)][0]
            we = wbuf[pl.ds(e, 16)][0]
            pe = parbuf[pl.ds(e, 16)][0]
            for k in range(D // 16):
                v = sb[e, pl.ds(k * 16, 16)]
                sb[e, pl.ds(k * 16, 16)] = v * se
            denbuf[e, pl.ds(0, 16)] = jnp.where(
                lane == pe, we, jnp.zeros((16,), jnp.float32))
            return cc
        lax.fori_loop(0, CHUNK, scale_body, 0)

        pltpu.sync_copy(sb, numer.at[di], add=True)
        pltpu.sync_copy(denbuf, den.at[didx2], add=True)

    issue_idx(0, 0)
    wait_idx(0, 0)
    issue_gathers(0)
    issue_idx(1, 1)

    def pair_body(j, carry):
        ca = 2 * j
        wait_gathers(0)
        wait_idx(ca + 1, 1)
        issue_gathers(1)
        compute_scatter(0)
        issue_idx(ca + 2, 0)
        wait_gathers(1)
        wait_idx(ca + 2, 0)
        issue_gathers(0)
        compute_scatter(1)

        @pl.when(ca + 3 < NCHUNK)
        def _issue_next():
            issue_idx(ca + 3, 1)
        return carry
    lax.fori_loop(0, (NCHUNK - 1) // 2, pair_body, 0)

    wait_gathers(0)
    compute_scatter(0)

    plsc.subcore_barrier()
    rows = pl.ds(s * RPT, RPT)
    pltpu.sync_copy(numer.at[rows], out_n.at[c, rows])
    drows = pl.ds(s * DRPT, DRPT)
    pltpu.sync_copy(den.at[drows], out_d.at[c, drows])

    @pl.when(s == 0)
    def _export_den_tail():
        tail = pl.ds(NS * DRPT, DEN_ROWS - NS * DRPT)
        pltpu.sync_copy(den.at[tail], out_d.at[c, tail])


def _make_edge_call():
    mesh = plsc.VectorSubcoreMesh(
        core_axis_name="c", subcore_axis_name="s",
        num_cores=NC, num_subcores=NS)
    return pl.kernel(
        _edge_kernel_body,
        out_type=(
            jax.ShapeDtypeStruct((NC, N_NODES, D), jnp.float32),
            jax.ShapeDtypeStruct((NC, DEN_ROWS, NPW), jnp.float32),
        ),
        mesh=mesh,
        compiler_params=pltpu.CompilerParams(
            use_tc_tiling_on_sc=False, needs_layout_passes=False),
        scratch_types=(
            [pltpu.VMEM((CHUNK, D), jnp.float32),
             pltpu.VMEM((CHUNK, D), jnp.float32),
             pltpu.VMEM((CHUNK, NPW), jnp.float32),
             pltpu.VMEM((CHUNK,), jnp.int32),
             pltpu.VMEM((CHUNK,), jnp.int32),
             ] * 2 +
            [pltpu.VMEM((CHUNK, NPW), jnp.float32),
             pltpu.VMEM((CHUNK,), jnp.int32),
             pltpu.VMEM((CHUNK + 16,), jnp.int32),
             pltpu.VMEM((16,), jnp.float32),
             pltpu.VMEM((CHUNK + 16,), jnp.float32),
             pltpu.VMEM((CHUNK + 16,), jnp.float32),
             pltpu.VMEM_SHARED((N_NODES, D), jnp.float32),
             pltpu.VMEM_SHARED((DEN_ROWS, NPW), jnp.float32),
             ] +
            [pltpu.SemaphoreType.DMA] * 10
        ),
    )


def kernel(x, edge_index, W1, b1, beta1, W2, b2, beta2):
    ei = edge_index.astype(jnp.int32)
    srce = ei[0]
    dste = ei[1]
    edge_call = _make_edge_call()

    hn1, np1 = _proj_call(x, W1, b1)
    pn1, pd1 = edge_call(hn1, np1, srce, dste,
                         jnp.full((16,), beta1, jnp.float32))
    pd1 = pd1.reshape(NC, N_NODES, 8)
    hn2, np2 = _combine_proj_call(pn1[0], pn1[1], pd1[0], pd1[1], W2, b2)
    pn2, pd2 = edge_call(hn2, np2, srce, dste,
                         jnp.full((16,), beta2, jnp.float32))
    pd2 = pd2.reshape(NC, N_NODES, 8)
    return _combine_out_call(pn2[0], pn2[1], pd2[0], pd2[1])


# speedup vs baseline: 1.2649x; 1.2649x over previous
import jax
import jax.numpy as jnp
from jax import lax
from jax.experimental import pallas as pl
from jax.experimental.pallas import tpu as pltpu
from jax.experimental.pallas import tpu_sc as plsc

N_NODES = 10000
N_EDGES = 320000
D = 128
NPW = 16
NC = 2
NS = 16
NW = NC * NS
EPT = N_EDGES // NW
CHUNK = 80
NCHUNK = EPT // CHUNK
RPT = N_NODES // NS
ZROWS = 125
DEN_ROWS = N_NODES // 2
DRPT = 312
BM = 1000


def _build_tables(h, hn_ref, np_ref):
    nrm = jnp.sqrt(jnp.sum(h * h, axis=1, keepdims=True)) + 1e-8
    hn_ref[...] = h / nrm
    lanes = lax.broadcasted_iota(jnp.int32, (h.shape[0], NPW), 1)
    np_ref[...] = jnp.where(lanes == 0, nrm, 0.0)


def _proj_table_kernel(x_ref, w_ref, b_ref, hn_ref, np_ref):
    h = jnp.dot(x_ref[...], w_ref[...],
                preferred_element_type=jnp.float32) + b_ref[...]
    _build_tables(h, hn_ref, np_ref)


def _combine(pn0, pn1, pd0, pd1):
    numer = pn0 + pn1
    den = (pd0 + pd1)[:, 0:1]
    return jnp.maximum(numer / (den + 1e-16), 0.0)


def _combine_proj_kernel(pn0_ref, pn1_ref, pd0_ref, pd1_ref, w_ref, b_ref,
                         hn_ref, np_ref):
    h1 = _combine(pn0_ref[...], pn1_ref[...], pd0_ref[...], pd1_ref[...])
    h = jnp.dot(h1, w_ref[...],
                preferred_element_type=jnp.float32) + b_ref[...]
    _build_tables(h, hn_ref, np_ref)


def _combine_out_kernel(pn0_ref, pn1_ref, pd0_ref, pd1_ref, out_ref):
    out_ref[...] = _combine(pn0_ref[...], pn1_ref[...],
                            pd0_ref[...], pd1_ref[...])


_TAB_OUT = (
    jax.ShapeDtypeStruct((N_NODES, D), jnp.float32),
    jax.ShapeDtypeStruct((N_NODES, NPW), jnp.float32),
)
_TAB_OUT_SPECS = (
    pl.BlockSpec((BM, D), lambda i: (i, 0)),
    pl.BlockSpec((BM, NPW), lambda i: (i, 0)),
)


def _proj_call(x, W, b):
    return pl.pallas_call(
        _proj_table_kernel,
        grid=(N_NODES // BM,),
        in_specs=[
            pl.BlockSpec((BM, D), lambda i: (i, 0)),
            pl.BlockSpec((D, D), lambda i: (0, 0)),
            pl.BlockSpec((1, D), lambda i: (0, 0)),
        ],
        out_specs=_TAB_OUT_SPECS,
        out_shape=_TAB_OUT,
    )(x, W, b.reshape(1, D))


def _combine_proj_call(pn0, pn1, pd0, pd1, W, b):
    return pl.pallas_call(
        _combine_proj_kernel,
        grid=(N_NODES // BM,),
        in_specs=[
            pl.BlockSpec((BM, D), lambda i: (i, 0)),
            pl.BlockSpec((BM, D), lambda i: (i, 0)),
            pl.BlockSpec((BM, 8), lambda i: (i, 0)),
            pl.BlockSpec((BM, 8), lambda i: (i, 0)),
            pl.BlockSpec((D, D), lambda i: (0, 0)),
            pl.BlockSpec((1, D), lambda i: (0, 0)),
        ],
        out_specs=_TAB_OUT_SPECS,
        out_shape=_TAB_OUT,
    )(pn0, pn1, pd0, pd1, W, b.reshape(1, D))


def _combine_out_call(pn0, pn1, pd0, pd1):
    return pl.pallas_call(
        _combine_out_kernel,
        grid=(N_NODES // BM,),
        in_specs=[
            pl.BlockSpec((BM, D), lambda i: (i, 0)),
            pl.BlockSpec((BM, D), lambda i: (i, 0)),
            pl.BlockSpec((BM, 8), lambda i: (i, 0)),
            pl.BlockSpec((BM, 8), lambda i: (i, 0)),
        ],
        out_specs=pl.BlockSpec((BM, D), lambda i: (i, 0)),
        out_shape=jax.ShapeDtypeStruct((N_NODES, D), jnp.float32),
    )(pn0, pn1, pd0, pd1)


def _edge_kernel_body(hn_tab, np_tab, srce, dste, betav, out_n, out_d,
                      srcbuf0, dstbuf0, npgbuf0, sidx0, didx0,
                      srcbuf1, dstbuf1, npgbuf1, sidx1, didx1,
                      denbuf, didx2, parbuf, betabuf, sbuf, wbuf,
                      numer, den,
                      gs0, gd0, gn0, gs1, gd1, gn1, is0, id0, is1, id1):
    c = lax.axis_index("c")
    s = lax.axis_index("s")
    tile_base = (c * NS + s) * EPT
    lane = lax.iota(jnp.int32, 16)
    SETS = (
        (srcbuf0, dstbuf0, npgbuf0, sidx0, didx0, gs0, gd0, gn0, is0, id0),
        (srcbuf1, dstbuf1, npgbuf1, sidx1, didx1, gs1, gd1, gn1, is1, id1),
    )

    pltpu.sync_copy(betav, betabuf)
    beta = betabuf[...]

    def _zero_body(i, carry):
        srcbuf0[i // (D // 16), pl.ds((i % (D // 16)) * 16, 16)] = (
            jnp.zeros((16,), jnp.float32))
        return carry
    lax.fori_loop(0, CHUNK * (D // 16), _zero_body, 0)

    def _dzero_body(i, carry):
        denbuf[i, pl.ds(0, 16)] = jnp.zeros((16,), jnp.float32)
        return carry
    lax.fori_loop(0, CHUNK, _dzero_body, 0)

    for k in range(RPT // CHUNK):
        pltpu.sync_copy(srcbuf0,
                        numer.at[pl.ds(s * RPT + k * CHUNK, CHUNK)])
    pltpu.sync_copy(srcbuf0.at[pl.ds(0, RPT % CHUNK)],
                    numer.at[pl.ds(s * RPT + (RPT // CHUNK) * CHUNK,
                                   RPT % CHUNK)])
    for k in range(DRPT // CHUNK):
        pltpu.sync_copy(denbuf, den.at[pl.ds(s * DRPT + k * CHUNK, CHUNK)])
    pltpu.sync_copy(denbuf.at[pl.ds(0, DRPT % CHUNK)],
                    den.at[pl.ds(s * DRPT + (DRPT // CHUNK) * CHUNK,
                                 DRPT % CHUNK)])

    @pl.when(s == 0)
    def _zero_den_tail():
        pltpu.sync_copy(denbuf.at[pl.ds(0, DEN_ROWS - NS * DRPT)],
                        den.at[pl.ds(NS * DRPT, DEN_ROWS - NS * DRPT)])

    plsc.subcore_barrier()


    def ebase(cv):
        return tile_base + cv * CHUNK

    def issue_idx(cv, p):
        _, _, _, si, di, _, _, _, i_s, i_d = SETS[p]
        pltpu.async_copy(srce.at[pl.ds(ebase(cv), CHUNK)], si, i_s)
        pltpu.async_copy(dste.at[pl.ds(ebase(cv), CHUNK)], di, i_d)

    def wait_idx(cv, p):
        _, _, _, si, di, _, _, _, i_s, i_d = SETS[p]
        pltpu.make_async_copy(srce.at[pl.ds(ebase(cv), CHUNK)], si, i_s).wait()
        pltpu.make_async_copy(dste.at[pl.ds(ebase(cv), CHUNK)], di, i_d).wait()

    def issue_gathers(p):
        sb, db, nb, si, di, g_s, g_d, g_n, _, _ = SETS[p]
        pltpu.async_copy(hn_tab.at[si], sb, g_s)
        pltpu.async_copy(hn_tab.at[di], db, g_d)
        pltpu.async_copy(np_tab.at[si], nb, g_n)

    def wait_gathers(p):
        sb, db, nb, si, di, g_s, g_d, g_n, _, _ = SETS[p]
        pltpu.make_async_copy(hn_tab.at[si], sb, g_s).wait()
        pltpu.make_async_copy(hn_tab.at[di], db, g_d).wait()
        pltpu.make_async_copy(np_tab.at[si], nb, g_n).wait()

    def compute_scatter(p):
        sb, db, nb, si, di, _, _, _, _, _ = SETS[p]

        for g in range(CHUNK // 16):
            dvi = di[pl.ds(g * 16, 16)]
            didx2[pl.ds(g * 16, 16)] = lax.shift_right_logical(dvi, 1)
            parbuf[pl.ds(g * 16, 16)] = (dvi & 1) * 8

        for g in range(CHUNK // 16):
            def dot16(e16, dv, g=g):
                e = g * 16 + e16
                prods = [sb[e, pl.ds(k * 16, 16)] * db[e, pl.ds(k * 16, 16)]
                         for k in range(D // 16)]
                while len(prods) > 1:
                    prods = [prods[i] + prods[i + 1]
                             for i in range(0, len(prods), 2)]
                return jnp.where(lane == e16, jnp.sum(prods[0]), dv)
            dv = lax.fori_loop(0, 16, dot16, jnp.zeros((16,), jnp.float32))
            rows = lane + g * 16
            npv = plsc.load_gather(
                nb, [rows, jnp.zeros((16,), jnp.int32)])
            wv = jnp.exp(beta * dv)
            sbuf[pl.ds(g * 16, 16)] = wv * npv
            wbuf[pl.ds(g * 16, 16)] = wv

        def scale_body(e, cc):
            se = sbuf[pl.ds(e, 16)][0]
            we = wbuf[pl.ds(e, 16)][0]
            pe = parbuf[pl.ds(e, 16)][0]
            for k in range(D // 16):
                v = sb[e, pl.ds(k * 16, 16)]
                sb[e, pl.ds(k * 16, 16)] = v * se
            denbuf[e, pl.ds(0, 16)] = jnp.where(
                lane == pe, we, jnp.zeros((16,), jnp.float32))
            return cc
        lax.fori_loop(0, CHUNK, scale_body, 0)

        pltpu.sync_copy(sb, numer.at[di], add=True)
        pltpu.sync_copy(denbuf, den.at[didx2], add=True)

    issue_idx(0, 0)
    wait_idx(0, 0)
    issue_gathers(0)
    issue_idx(1, 1)

    def pair_body(j, carry):
        ca = 2 * j
        wait_gathers(0)
        wait_idx(ca + 1, 1)
        issue_gathers(1)
        compute_scatter(0)
        issue_idx(ca + 2, 0)
        wait_gathers(1)
        wait_idx(ca + 2, 0)
        issue_gathers(0)
        compute_scatter(1)

        @pl.when(ca + 3 < NCHUNK)
        def _issue_next():
            issue_idx(ca + 3, 1)
        return carry
    lax.fori_loop(0, (NCHUNK - 1) // 2, pair_body, 0)

    wait_gathers(0)
    compute_scatter(0)

    plsc.subcore_barrier()
    rows = pl.ds(s * RPT, RPT)
    pltpu.sync_copy(numer.at[rows], out_n.at[c, rows])
    drows = pl.ds(s * DRPT, DRPT)
    pltpu.sync_copy(den.at[drows], out_d.at[c, drows])

    @pl.when(s == 0)
    def _export_den_tail():
        tail = pl.ds(NS * DRPT, DEN_ROWS - NS * DRPT)
        pltpu.sync_copy(den.at[tail], out_d.at[c, tail])


def _make_edge_call():
    mesh = plsc.VectorSubcoreMesh(
        core_axis_name="c", subcore_axis_name="s",
        num_cores=NC, num_subcores=NS)
    return pl.kernel(
        _edge_kernel_body,
        out_type=(
            jax.ShapeDtypeStruct((NC, N_NODES, D), jnp.float32),
            jax.ShapeDtypeStruct((NC, DEN_ROWS, NPW), jnp.float32),
        ),
        mesh=mesh,
        compiler_params=pltpu.CompilerParams(
            use_tc_tiling_on_sc=False, needs_layout_passes=False),
        scratch_types=(
            [pltpu.VMEM((CHUNK, D), jnp.float32),
             pltpu.VMEM((CHUNK, D), jnp.float32),
             pltpu.VMEM((CHUNK, NPW), jnp.float32),
             pltpu.VMEM((CHUNK,), jnp.int32),
             pltpu.VMEM((CHUNK,), jnp.int32),
             ] * 2 +
            [pltpu.VMEM((CHUNK, NPW), jnp.float32),
             pltpu.VMEM((CHUNK,), jnp.int32),
             pltpu.VMEM((CHUNK + 16,), jnp.int32),
             pltpu.VMEM((16,), jnp.float32),
             pltpu.VMEM((CHUNK + 16,), jnp.float32),
             pltpu.VMEM((CHUNK + 16,), jnp.float32),
             pltpu.VMEM_SHARED((N_NODES, D), jnp.float32),
             pltpu.VMEM_SHARED((DEN_ROWS, NPW), jnp.float32),
             ] +
            [pltpu.SemaphoreType.DMA] * 10
        ),
    )


def kernel(x, edge_index, W1, b1, beta1, W2, b2, beta2):
    ei = edge_index.astype(jnp.int32)
    srce = ei[0]
    dste = ei[1]
    edge_call = _make_edge_call()

    hn1, np1 = _proj_call(x, W1, b1)
    pn1, pd1 = edge_call(hn1, np1, srce, dste,
                         jnp.full((16,), beta1, jnp.float32))
    pd1 = pd1.reshape(NC, N_NODES, 8)
    hn2, np2 = _combine_proj_call(pn1[0], pn1[1], pd1[0], pd1[1], W2, b2)
    pn2, pd2 = edge_call(hn2, np2, srce, dste,
                         jnp.full((16,), beta2, jnp.float32))
    pd2 = pd2.reshape(NC, N_NODES, 8)
    return _combine_out_call(pn2[0], pn2[1], pd2[0], pd2[1])


# speedup vs baseline: 1.2769x; 1.0095x over previous
import jax
import jax.numpy as jnp
from jax import lax
from jax.experimental import pallas as pl
from jax.experimental.pallas import tpu as pltpu
from jax.experimental.pallas import tpu_sc as plsc

N_NODES = 10000
N_EDGES = 320000
D = 128
NPW = 16
NC = 2
NS = 16
NW = NC * NS
EPT = N_EDGES // NW
CHUNK = 80
NCHUNK = EPT // CHUNK
RPT = N_NODES // NS
ZROWS = 125
DEN_ROWS = N_NODES // 2
DRPT = 312
BM = 1000


def _build_tables(h, hn_ref, np_ref):
    nrm = jnp.sqrt(jnp.sum(h * h, axis=1, keepdims=True)) + 1e-8
    hn_ref[...] = h / nrm
    lanes = lax.broadcasted_iota(jnp.int32, (h.shape[0], NPW), 1)
    np_ref[...] = jnp.where(lanes == 0, nrm, 0.0)


def _proj_table_kernel(x_ref, w_ref, b_ref, hn_ref, np_ref):
    h = jnp.dot(x_ref[...], w_ref[...],
                preferred_element_type=jnp.float32) + b_ref[...]
    _build_tables(h, hn_ref, np_ref)


def _combine(pn0, pn1, pd0, pd1):
    numer = pn0 + pn1
    den = (pd0 + pd1)[:, 0:1]
    return jnp.maximum(numer / (den + 1e-16), 0.0)


def _combine_proj_kernel(pn0_ref, pn1_ref, pd0_ref, pd1_ref, w_ref, b_ref,
                         hn_ref, np_ref):
    h1 = _combine(pn0_ref[...], pn1_ref[...], pd0_ref[...], pd1_ref[...])
    h = jnp.dot(h1, w_ref[...],
                preferred_element_type=jnp.float32) + b_ref[...]
    _build_tables(h, hn_ref, np_ref)


def _combine_out_kernel(pn0_ref, pn1_ref, pd0_ref, pd1_ref, out_ref):
    out_ref[...] = _combine(pn0_ref[...], pn1_ref[...],
                            pd0_ref[...], pd1_ref[...])


_TAB_OUT = (
    jax.ShapeDtypeStruct((N_NODES, D), jnp.float32),
    jax.ShapeDtypeStruct((N_NODES, NPW), jnp.float32),
)
_TAB_OUT_SPECS = (
    pl.BlockSpec((BM, D), lambda i: (i, 0)),
    pl.BlockSpec((BM, NPW), lambda i: (i, 0)),
)


def _proj_call(x, W, b):
    return pl.pallas_call(
        _proj_table_kernel,
        grid=(N_NODES // BM,),
        in_specs=[
            pl.BlockSpec((BM, D), lambda i: (i, 0)),
            pl.BlockSpec((D, D), lambda i: (0, 0)),
            pl.BlockSpec((1, D), lambda i: (0, 0)),
        ],
        out_specs=_TAB_OUT_SPECS,
        out_shape=_TAB_OUT,
    )(x, W, b.reshape(1, D))


def _combine_proj_call(pn0, pn1, pd0, pd1, W, b):
    return pl.pallas_call(
        _combine_proj_kernel,
        grid=(N_NODES // BM,),
        in_specs=[
            pl.BlockSpec((BM, D), lambda i: (i, 0)),
            pl.BlockSpec((BM, D), lambda i: (i, 0)),
            pl.BlockSpec((BM, 8), lambda i: (i, 0)),
            pl.BlockSpec((BM, 8), lambda i: (i, 0)),
            pl.BlockSpec((D, D), lambda i: (0, 0)),
            pl.BlockSpec((1, D), lambda i: (0, 0)),
        ],
        out_specs=_TAB_OUT_SPECS,
        out_shape=_TAB_OUT,
    )(pn0, pn1, pd0, pd1, W, b.reshape(1, D))


def _combine_out_call(pn0, pn1, pd0, pd1):
    return pl.pallas_call(
        _combine_out_kernel,
        grid=(N_NODES // BM,),
        in_specs=[
            pl.BlockSpec((BM, D), lambda i: (i, 0)),
            pl.BlockSpec((BM, D), lambda i: (i, 0)),
            pl.BlockSpec((BM, 8), lambda i: (i, 0)),
            pl.BlockSpec((BM, 8), lambda i: (i, 0)),
        ],
        out_specs=pl.BlockSpec((BM, D), lambda i: (i, 0)),
        out_shape=jax.ShapeDtypeStruct((N_NODES, D), jnp.float32),
    )(pn0, pn1, pd0, pd1)


def _edge_kernel_body(hn_tab, np_tab, srce, dste, betav, out_n, out_d,
                      srcbuf0, dstbuf0, npgbuf0, sidx0, didx0,
                      srcbuf1, dstbuf1, npgbuf1, sidx1, didx1,
                      denbuf, didx2, betabuf, sbuf,
                      numer, den,
                      gs0, gd0, gn0, gs1, gd1, gn1, is0, id0, is1, id1):
    c = lax.axis_index("c")
    s = lax.axis_index("s")
    tile_base = (c * NS + s) * EPT
    lane = lax.iota(jnp.int32, 16)
    SETS = (
        (srcbuf0, dstbuf0, npgbuf0, sidx0, didx0, gs0, gd0, gn0, is0, id0),
        (srcbuf1, dstbuf1, npgbuf1, sidx1, didx1, gs1, gd1, gn1, is1, id1),
    )

    pltpu.sync_copy(betav, betabuf)
    beta = betabuf[...]

    def _zero_body(i, carry):
        srcbuf0[i // (D // 16), pl.ds((i % (D // 16)) * 16, 16)] = (
            jnp.zeros((16,), jnp.float32))
        return carry
    lax.fori_loop(0, CHUNK * (D // 16), _zero_body, 0)

    def _dzero_body(i, carry):
        denbuf[i, pl.ds(0, 16)] = jnp.zeros((16,), jnp.float32)
        return carry
    lax.fori_loop(0, CHUNK, _dzero_body, 0)

    for k in range(RPT // CHUNK):
        pltpu.sync_copy(srcbuf0,
                        numer.at[pl.ds(s * RPT + k * CHUNK, CHUNK)])
    pltpu.sync_copy(srcbuf0.at[pl.ds(0, RPT % CHUNK)],
                    numer.at[pl.ds(s * RPT + (RPT // CHUNK) * CHUNK,
                                   RPT % CHUNK)])
    for k in range(DRPT // CHUNK):
        pltpu.sync_copy(denbuf, den.at[pl.ds(s * DRPT + k * CHUNK, CHUNK)])
    pltpu.sync_copy(denbuf.at[pl.ds(0, DRPT % CHUNK)],
                    den.at[pl.ds(s * DRPT + (DRPT // CHUNK) * CHUNK,
                                 DRPT % CHUNK)])

    @pl.when(s == 0)
    def _zero_den_tail():
        pltpu.sync_copy(denbuf.at[pl.ds(0, DEN_ROWS - NS * DRPT)],
                        den.at[pl.ds(NS * DRPT, DEN_ROWS - NS * DRPT)])

    plsc.subcore_barrier()


    def ebase(cv):
        return tile_base + cv * CHUNK

    def issue_idx(cv, p):
        _, _, _, si, di, _, _, _, i_s, i_d = SETS[p]
        pltpu.async_copy(srce.at[pl.ds(ebase(cv), CHUNK)], si, i_s)
        pltpu.async_copy(dste.at[pl.ds(ebase(cv), CHUNK)], di, i_d)

    def wait_idx(cv, p):
        _, _, _, si, di, _, _, _, i_s, i_d = SETS[p]
        pltpu.make_async_copy(srce.at[pl.ds(ebase(cv), CHUNK)], si, i_s).wait()
        pltpu.make_async_copy(dste.at[pl.ds(ebase(cv), CHUNK)], di, i_d).wait()

    def issue_gathers(p):
        sb, db, nb, si, di, g_s, g_d, g_n, _, _ = SETS[p]
        pltpu.async_copy(hn_tab.at[si], sb, g_s)
        pltpu.async_copy(hn_tab.at[di], db, g_d)
        pltpu.async_copy(np_tab.at[si], nb, g_n)

    def wait_gathers(p):
        sb, db, nb, si, di, g_s, g_d, g_n, _, _ = SETS[p]
        pltpu.make_async_copy(hn_tab.at[si], sb, g_s).wait()
        pltpu.make_async_copy(hn_tab.at[di], db, g_d).wait()
        pltpu.make_async_copy(np_tab.at[si], nb, g_n).wait()

    def compute_scatter(p):
        sb, db, nb, si, di, _, _, _, _, _ = SETS[p]

        for g in range(CHUNK // 16):
            dvi = di[pl.ds(g * 16, 16)]
            didx2[pl.ds(g * 16, 16)] = lax.shift_right_logical(dvi, 1)

        for g in range(CHUNK // 16):
            def dot16(e16, dv, g=g):
                e = g * 16 + e16
                prods = [sb[e, pl.ds(k * 16, 16)] * db[e, pl.ds(k * 16, 16)]
                         for k in range(D // 16)]
                while len(prods) > 1:
                    prods = [prods[i] + prods[i + 1]
                             for i in range(0, len(prods), 2)]
                return jnp.where(lane == e16, jnp.sum(prods[0]), dv)
            dv = lax.fori_loop(0, 16, dot16, jnp.zeros((16,), jnp.float32))
            rows = lane + g * 16
            npv = plsc.load_gather(
                nb, [rows, jnp.zeros((16,), jnp.int32)])
            wv = jnp.exp(beta * dv)
            sbuf[pl.ds(g * 16, 16)] = wv * npv
            dvi = di[pl.ds(g * 16, 16)]
            pev = (dvi & 1) * 8
            plsc.store_scatter(denbuf, [rows, pev ^ 8],
                               jnp.zeros((16,), jnp.float32))
            plsc.store_scatter(denbuf, [rows, pev], wv)

        def scale_body(e, cc):
            se = sbuf[pl.ds(e, 16)][0]
            for k in range(D // 16):
                v = sb[e, pl.ds(k * 16, 16)]
                sb[e, pl.ds(k * 16, 16)] = v * se
            return cc
        lax.fori_loop(0, CHUNK, scale_body, 0)

        pltpu.sync_copy(sb, numer.at[di], add=True)
        pltpu.sync_copy(denbuf, den.at[didx2], add=True)

    issue_idx(0, 0)
    wait_idx(0, 0)
    issue_gathers(0)
    issue_idx(1, 1)

    def pair_body(j, carry):
        ca = 2 * j
        wait_gathers(0)
        wait_idx(ca + 1, 1)
        issue_gathers(1)
        compute_scatter(0)
        issue_idx(ca + 2, 0)
        wait_gathers(1)
        wait_idx(ca + 2, 0)
        issue_gathers(0)
        compute_scatter(1)

        @pl.when(ca + 3 < NCHUNK)
        def _issue_next():
            issue_idx(ca + 3, 1)
        return carry
    lax.fori_loop(0, (NCHUNK - 1) // 2, pair_body, 0)

    wait_gathers(0)
    compute_scatter(0)

    plsc.subcore_barrier()
    rows = pl.ds(s * RPT, RPT)
    pltpu.sync_copy(numer.at[rows], out_n.at[c, rows])
    drows = pl.ds(s * DRPT, DRPT)
    pltpu.sync_copy(den.at[drows], out_d.at[c, drows])

    @pl.when(s == 0)
    def _export_den_tail():
        tail = pl.ds(NS * DRPT, DEN_ROWS - NS * DRPT)
        pltpu.sync_copy(den.at[tail], out_d.at[c, tail])


def _make_edge_call():
    mesh = plsc.VectorSubcoreMesh(
        core_axis_name="c", subcore_axis_name="s",
        num_cores=NC, num_subcores=NS)
    return pl.kernel(
        _edge_kernel_body,
        out_type=(
            jax.ShapeDtypeStruct((NC, N_NODES, D), jnp.float32),
            jax.ShapeDtypeStruct((NC, DEN_ROWS, NPW), jnp.float32),
        ),
        mesh=mesh,
        compiler_params=pltpu.CompilerParams(
            use_tc_tiling_on_sc=False, needs_layout_passes=False),
        scratch_types=(
            [pltpu.VMEM((CHUNK, D), jnp.float32),
             pltpu.VMEM((CHUNK, D), jnp.float32),
             pltpu.VMEM((CHUNK, NPW), jnp.float32),
             pltpu.VMEM((CHUNK,), jnp.int32),
             pltpu.VMEM((CHUNK,), jnp.int32),
             ] * 2 +
            [pltpu.VMEM((CHUNK, NPW), jnp.float32),
             pltpu.VMEM((CHUNK,), jnp.int32),
             pltpu.VMEM((16,), jnp.float32),
             pltpu.VMEM((CHUNK + 16,), jnp.float32),
             pltpu.VMEM_SHARED((N_NODES, D), jnp.float32),
             pltpu.VMEM_SHARED((DEN_ROWS, NPW), jnp.float32),
             ] +
            [pltpu.SemaphoreType.DMA] * 10
        ),
    )


def kernel(x, edge_index, W1, b1, beta1, W2, b2, beta2):
    ei = edge_index.astype(jnp.int32)
    srce = ei[0]
    dste = ei[1]
    edge_call = _make_edge_call()

    hn1, np1 = _proj_call(x, W1, b1)
    pn1, pd1 = edge_call(hn1, np1, srce, dste,
                         jnp.full((16,), beta1, jnp.float32))
    pd1 = pd1.reshape(NC, N_NODES, 8)
    hn2, np2 = _combine_proj_call(pn1[0], pn1[1], pd1[0], pd1[1], W2, b2)
    pn2, pd2 = edge_call(hn2, np2, srce, dste,
                         jnp.full((16,), beta2, jnp.float32))
    pd2 = pd2.reshape(NC, N_NODES, 8)
    return _combine_out_call(pn2[0], pn2[1], pd2[0], pd2[1])


# speedup vs baseline: 1.2789x; 1.0016x over previous
import jax
import jax.numpy as jnp
from jax import lax
from jax.experimental import pallas as pl
from jax.experimental.pallas import tpu as pltpu
from jax.experimental.pallas import tpu_sc as plsc

N_NODES = 10000
N_EDGES = 320000
D = 128
NPW = 16
NC = 2
NS = 16
NW = NC * NS
EPT = N_EDGES // NW
CHUNK = 80
NCHUNK = EPT // CHUNK
RPT = N_NODES // NS
ZROWS = 125
DEN_ROWS = N_NODES // 2
DRPT = 312
BM = 1000


def _build_tables(h, hn_ref, np_ref):
    nrm = jnp.sqrt(jnp.sum(h * h, axis=1, keepdims=True)) + 1e-8
    hn_ref[...] = h / nrm
    lanes = lax.broadcasted_iota(jnp.int32, (h.shape[0], NPW), 1)
    np_ref[...] = jnp.where(lanes == 0, nrm, 0.0)


def _proj_table_kernel(x_ref, w_ref, b_ref, hn_ref, np_ref):
    h = jnp.dot(x_ref[...], w_ref[...],
                preferred_element_type=jnp.float32) + b_ref[...]
    _build_tables(h, hn_ref, np_ref)


def _combine(pn0, pn1, pd0, pd1):
    numer = pn0 + pn1
    den = (pd0 + pd1)[:, 0:1]
    return jnp.maximum(numer / (den + 1e-16), 0.0)


def _combine_proj_kernel(pn0_ref, pn1_ref, pd0_ref, pd1_ref, w_ref, b_ref,
                         hn_ref, np_ref):
    h1 = _combine(pn0_ref[...], pn1_ref[...], pd0_ref[...], pd1_ref[...])
    h = jnp.dot(h1, w_ref[...],
                preferred_element_type=jnp.float32) + b_ref[...]
    _build_tables(h, hn_ref, np_ref)


def _combine_out_kernel(pn0_ref, pn1_ref, pd0_ref, pd1_ref, out_ref):
    out_ref[...] = _combine(pn0_ref[...], pn1_ref[...],
                            pd0_ref[...], pd1_ref[...])


_TAB_OUT = (
    jax.ShapeDtypeStruct((N_NODES, D), jnp.float32),
    jax.ShapeDtypeStruct((N_NODES, NPW), jnp.float32),
)
_TAB_OUT_SPECS = (
    pl.BlockSpec((BM, D), lambda i: (i, 0)),
    pl.BlockSpec((BM, NPW), lambda i: (i, 0)),
)


def _proj_call(x, W, b):
    return pl.pallas_call(
        _proj_table_kernel,
        grid=(N_NODES // BM,),
        in_specs=[
            pl.BlockSpec((BM, D), lambda i: (i, 0)),
            pl.BlockSpec((D, D), lambda i: (0, 0)),
            pl.BlockSpec((1, D), lambda i: (0, 0)),
        ],
        out_specs=_TAB_OUT_SPECS,
        out_shape=_TAB_OUT,
    )(x, W, b.reshape(1, D))


def _combine_proj_call(pn0, pn1, pd0, pd1, W, b):
    return pl.pallas_call(
        _combine_proj_kernel,
        grid=(N_NODES // BM,),
        in_specs=[
            pl.BlockSpec((BM, D), lambda i: (i, 0)),
            pl.BlockSpec((BM, D), lambda i: (i, 0)),
            pl.BlockSpec((BM, 8), lambda i: (i, 0)),
            pl.BlockSpec((BM, 8), lambda i: (i, 0)),
            pl.BlockSpec((D, D), lambda i: (0, 0)),
            pl.BlockSpec((1, D), lambda i: (0, 0)),
        ],
        out_specs=_TAB_OUT_SPECS,
        out_shape=_TAB_OUT,
    )(pn0, pn1, pd0, pd1, W, b.reshape(1, D))


def _combine_out_call(pn0, pn1, pd0, pd1):
    return pl.pallas_call(
        _combine_out_kernel,
        grid=(N_NODES // BM,),
        in_specs=[
            pl.BlockSpec((BM, D), lambda i: (i, 0)),
            pl.BlockSpec((BM, D), lambda i: (i, 0)),
            pl.BlockSpec((BM, 8), lambda i: (i, 0)),
            pl.BlockSpec((BM, 8), lambda i: (i, 0)),
        ],
        out_specs=pl.BlockSpec((BM, D), lambda i: (i, 0)),
        out_shape=jax.ShapeDtypeStruct((N_NODES, D), jnp.float32),
    )(pn0, pn1, pd0, pd1)


def _edge_kernel_body(hn_tab, tbf_tab, np_tab, srce, dste, betav, out_n, out_d,
                      srcbuf0, dstbuf0, npgbuf0, sidx0, didx0,
                      srcbuf1, dstbuf1, npgbuf1, sidx1, didx1,
                      denbuf, didx2, betabuf, sbuf,
                      numer, den,
                      gs0, gd0, gn0, gs1, gd1, gn1, is0, id0, is1, id1):
    c = lax.axis_index("c")
    s = lax.axis_index("s")
    tile_base = (c * NS + s) * EPT
    lane = lax.iota(jnp.int32, 16)
    SETS = (
        (srcbuf0, dstbuf0, npgbuf0, sidx0, didx0, gs0, gd0, gn0, is0, id0),
        (srcbuf1, dstbuf1, npgbuf1, sidx1, didx1, gs1, gd1, gn1, is1, id1),
    )

    pltpu.sync_copy(betav, betabuf)
    beta = betabuf[...]

    def _zero_body(i, carry):
        srcbuf0[i // (D // 16), pl.ds((i % (D // 16)) * 16, 16)] = (
            jnp.zeros((16,), jnp.float32))
        return carry
    lax.fori_loop(0, CHUNK * (D // 16), _zero_body, 0)

    def _dzero_body(i, carry):
        denbuf[i, pl.ds(0, 16)] = jnp.zeros((16,), jnp.float32)
        return carry
    lax.fori_loop(0, CHUNK, _dzero_body, 0)

    for k in range(RPT // CHUNK):
        pltpu.sync_copy(srcbuf0,
                        numer.at[pl.ds(s * RPT + k * CHUNK, CHUNK)])
    pltpu.sync_copy(srcbuf0.at[pl.ds(0, RPT % CHUNK)],
                    numer.at[pl.ds(s * RPT + (RPT // CHUNK) * CHUNK,
                                   RPT % CHUNK)])
    for k in range(DRPT // CHUNK):
        pltpu.sync_copy(denbuf, den.at[pl.ds(s * DRPT + k * CHUNK, CHUNK)])
    pltpu.sync_copy(denbuf.at[pl.ds(0, DRPT % CHUNK)],
                    den.at[pl.ds(s * DRPT + (DRPT // CHUNK) * CHUNK,
                                 DRPT % CHUNK)])

    @pl.when(s == 0)
    def _zero_den_tail():
        pltpu.sync_copy(denbuf.at[pl.ds(0, DEN_ROWS - NS * DRPT)],
                        den.at[pl.ds(NS * DRPT, DEN_ROWS - NS * DRPT)])

    plsc.subcore_barrier()


    def ebase(cv):
        return tile_base + cv * CHUNK

    def issue_idx(cv, p):
        _, _, _, si, di, _, _, _, i_s, i_d = SETS[p]
        pltpu.async_copy(srce.at[pl.ds(ebase(cv), CHUNK)], si, i_s)
        pltpu.async_copy(dste.at[pl.ds(ebase(cv), CHUNK)], di, i_d)

    def wait_idx(cv, p):
        _, _, _, si, di, _, _, _, i_s, i_d = SETS[p]
        pltpu.make_async_copy(srce.at[pl.ds(ebase(cv), CHUNK)], si, i_s).wait()
        pltpu.make_async_copy(dste.at[pl.ds(ebase(cv), CHUNK)], di, i_d).wait()

    def issue_gathers(p):
        sb, db, nb, si, di, g_s, g_d, g_n, _, _ = SETS[p]
        pltpu.async_copy(hn_tab.at[si], sb, g_s)
        pltpu.async_copy(tbf_tab.at[di], db, g_d)
        pltpu.async_copy(np_tab.at[si], nb, g_n)

    def wait_gathers(p):
        sb, db, nb, si, di, g_s, g_d, g_n, _, _ = SETS[p]
        pltpu.make_async_copy(hn_tab.at[si], sb, g_s).wait()
        pltpu.make_async_copy(tbf_tab.at[di], db, g_d).wait()
        pltpu.make_async_copy(np_tab.at[si], nb, g_n).wait()

    def compute_scatter(p):
        sb, db, nb, si, di, _, _, _, _, _ = SETS[p]

        for g in range(CHUNK // 16):
            dvi = di[pl.ds(g * 16, 16)]
            didx2[pl.ds(g * 16, 16)] = lax.shift_right_logical(dvi, 1)

        for g in range(CHUNK // 16):
            def dot16(e16, dv, g=g):
                e = g * 16 + e16
                prods = []
                for k2 in range(D // 32):
                    v = db[e, pl.ds(k2 * 32, 32)]
                    a, b = plsc.unpack(v, format=plsc.PackFormat.INTERLEAVED)
                    prods.append(sb[e, pl.ds((2 * k2) * 16, 16)] * a)
                    prods.append(sb[e, pl.ds((2 * k2 + 1) * 16, 16)] * b)
                while len(prods) > 1:
                    prods = [prods[i] + prods[i + 1]
                             for i in range(0, len(prods), 2)]
                return jnp.where(lane == e16, jnp.sum(prods[0]), dv)
            dv = lax.fori_loop(0, 16, dot16, jnp.zeros((16,), jnp.float32))
            rows = lane + g * 16
            npv = plsc.load_gather(
                nb, [rows, jnp.zeros((16,), jnp.int32)])
            wv = jnp.exp(beta * dv)
            sbuf[pl.ds(g * 16, 16)] = wv * npv
            dvi = di[pl.ds(g * 16, 16)]
            pev = (dvi & 1) * 8
            plsc.store_scatter(denbuf, [rows, pev ^ 8],
                               jnp.zeros((16,), jnp.float32))
            plsc.store_scatter(denbuf, [rows, pev], wv)

        def scale_body(e, cc):
            se = sbuf[pl.ds(e, 16)][0]
            for k in range(D // 16):
                v = sb[e, pl.ds(k * 16, 16)]
                sb[e, pl.ds(k * 16, 16)] = v * se
            return cc
        lax.fori_loop(0, CHUNK, scale_body, 0)

        pltpu.sync_copy(sb, numer.at[di], add=True)
        pltpu.sync_copy(denbuf, den.at[didx2], add=True)

    issue_idx(0, 0)
    wait_idx(0, 0)
    issue_gathers(0)
    issue_idx(1, 1)

    def pair_body(j, carry):
        ca = 2 * j
        wait_gathers(0)
        wait_idx(ca + 1, 1)
        issue_gathers(1)
        compute_scatter(0)
        issue_idx(ca + 2, 0)
        wait_gathers(1)
        wait_idx(ca + 2, 0)
        issue_gathers(0)
        compute_scatter(1)

        @pl.when(ca + 3 < NCHUNK)
        def _issue_next():
            issue_idx(ca + 3, 1)
        return carry
    lax.fori_loop(0, (NCHUNK - 1) // 2, pair_body, 0)

    wait_gathers(0)
    compute_scatter(0)

    plsc.subcore_barrier()
    rows = pl.ds(s * RPT, RPT)
    pltpu.sync_copy(numer.at[rows], out_n.at[c, rows])
    drows = pl.ds(s * DRPT, DRPT)
    pltpu.sync_copy(den.at[drows], out_d.at[c, drows])

    @pl.when(s == 0)
    def _export_den_tail():
        tail = pl.ds(NS * DRPT, DEN_ROWS - NS * DRPT)
        pltpu.sync_copy(den.at[tail], out_d.at[c, tail])


def _make_edge_call():
    mesh = plsc.VectorSubcoreMesh(
        core_axis_name="c", subcore_axis_name="s",
        num_cores=NC, num_subcores=NS)
    return pl.kernel(
        _edge_kernel_body,
        out_type=(
            jax.ShapeDtypeStruct((NC, N_NODES, D), jnp.float32),
            jax.ShapeDtypeStruct((NC, DEN_ROWS, NPW), jnp.float32),
        ),
        name="agnn_edge_softmax_scatter",
        mesh=mesh,
        compiler_params=pltpu.CompilerParams(
            use_tc_tiling_on_sc=False, needs_layout_passes=False),
        scratch_types=(
            [pltpu.VMEM((CHUNK, D), jnp.float32),
             pltpu.VMEM((CHUNK, D), jnp.bfloat16),
             pltpu.VMEM((CHUNK, NPW), jnp.float32),
             pltpu.VMEM((CHUNK,), jnp.int32),
             pltpu.VMEM((CHUNK,), jnp.int32),
             ] * 2 +
            [pltpu.VMEM((CHUNK, NPW), jnp.float32),
             pltpu.VMEM((CHUNK,), jnp.int32),
             pltpu.VMEM((16,), jnp.float32),
             pltpu.VMEM((CHUNK + 16,), jnp.float32),
             pltpu.VMEM_SHARED((N_NODES, D), jnp.float32),
             pltpu.VMEM_SHARED((DEN_ROWS, NPW), jnp.float32),
             ] +
            [pltpu.SemaphoreType.DMA] * 10
        ),
    )


def _bf16_interleaved(hn):
    x = hn.reshape(N_NODES, D // 32, 2, 16)
    return jnp.swapaxes(x, 2, 3).reshape(N_NODES, D).astype(jnp.bfloat16)


def kernel(x, edge_index, W1, b1, beta1, W2, b2, beta2):
    ei = edge_index.astype(jnp.int32)
    srce = ei[0]
    dste = ei[1]
    edge_call = _make_edge_call()

    hn1, np1 = _proj_call(x, W1, b1)
    pn1, pd1 = edge_call(hn1, _bf16_interleaved(hn1), np1, srce, dste,
                         jnp.full((16,), beta1, jnp.float32))
    pd1 = pd1.reshape(NC, N_NODES, 8)
    hn2, np2 = _combine_proj_call(pn1[0], pn1[1], pd1[0], pd1[1], W2, b2)
    pn2, pd2 = edge_call(hn2, _bf16_interleaved(hn2), np2, srce, dste,
                         jnp.full((16,), beta2, jnp.float32))
    pd2 = pd2.reshape(NC, N_NODES, 8)
    return _combine_out_call(pn2[0], pn2[1], pd2[0], pd2[1])
